# baseline (speedup 1.0000x reference)
import functools

import jax
import jax.numpy as jnp
from jax.experimental import pallas as pl
from jax.experimental.pallas import tpu as pltpu

KH, KW = 32, 32
STRIDE = 4
PAD = 10
H, W, C = 224, 224, 3
DIM = KH * KW * C
N_MEM = 4096
OH = (H + 2 * PAD - KH) // STRIDE + 1
OW = (W + 2 * PAD - KW) // STRIDE + 1
Q = OH * OW
U = (H + 2 * PAD) // STRIDE
AR = KH // STRIDE
G = C * KH
OWP = 56
QP = OH * OWP

TQ = 1008
TK = 512
KC = 512


def _unfold_kernel(img_ref, q_ref, qsq_ref):
    i = pl.program_id(0)
    blk = img_ref[:, pl.ds(i, AR), :, :]
    rm = blk.reshape(G, H + 2 * PAD)
    parts = [rm[:, 4 * min(j, OW - 1) : 4 * min(j, OW - 1) + KW]
             for j in range(OWP)]
    q3 = jnp.stack(parts, axis=0)
    q_ref[0] = q3
    qsq_ref[0] = jnp.sum(q3 * q3, axis=(1, 2), keepdims=False)[:, None]


def _search_kernel(q_ref, qsq_ref, k_ref, dist_ref, idx_ref, ksq_ref):
    qi = pl.program_id(0)
    ki = pl.program_id(1)
    kt = k_ref[...]

    @pl.when(qi == 0)
    def _fill_ksq():
        ksq_ref[ki] = jnp.sum(kt * kt, axis=1)[None, :]

    ksq = ksq_ref[ki]
    dot = jnp.zeros((TQ, TK), jnp.float32)
    for c in range(0, DIM, KC):
        dot = dot + jax.lax.dot_general(
            q_ref[:, c : c + KC], kt[:, c : c + KC],
            dimension_numbers=(((1,), (1,)), ((), ())),
            precision=jax.lax.Precision.DEFAULT,
            preferred_element_type=jnp.float32,
        )
    d = (qsq_ref[...] + ksq) - 2.0 * dot
    m = jnp.min(d, axis=1, keepdims=True)
    iota = jax.lax.broadcasted_iota(jnp.int32, d.shape, 1) + ki * TK
    am = jnp.min(jnp.where(d == m, iota, jnp.int32(2**30)), axis=1,
                 keepdims=True)

    @pl.when(ki == 0)
    def _init():
        dist_ref[...] = m
        idx_ref[...] = am

    @pl.when(ki != 0)
    def _update():
        better = m < dist_ref[...]
        idx_ref[...] = jnp.where(better, am, idx_ref[...])
        dist_ref[...] = jnp.where(better, m, dist_ref[...])


@functools.partial(jax.jit, static_argnames=("interpret",))
def kernel(image, mem_keys, interpret=False):
    img = jnp.transpose(image, (2, 0, 1))
    img = jnp.pad(img, ((0, 0), (PAD, PAD), (PAD, PAD)))
    img4 = img.reshape(C, U, STRIDE, H + 2 * PAD)

    q, qsq = pl.pallas_call(
        _unfold_kernel,
        grid=(OH,),
        in_specs=[pl.BlockSpec(img4.shape, lambda i: (0, 0, 0, 0))],
        out_specs=[
            pl.BlockSpec((1, OWP, G, KW), lambda i: (i, 0, 0, 0)),
            pl.BlockSpec((1, OWP, 1), lambda i: (i, 0, 0)),
        ],
        out_shape=[
            jax.ShapeDtypeStruct((OH, OWP, G, KW), jnp.float32),
            jax.ShapeDtypeStruct((OH, OWP, 1), jnp.float32),
        ],
        interpret=interpret,
    )(img4)
    q = q.reshape(QP, DIM)
    qsq = qsq.reshape(QP, 1)

    dists, idx = pl.pallas_call(
        _search_kernel,
        grid=(QP // TQ, N_MEM // TK),
        in_specs=[
            pl.BlockSpec((TQ, DIM), lambda qi, ki: (qi, 0)),
            pl.BlockSpec((TQ, 1), lambda qi, ki: (qi, 0)),
            pl.BlockSpec((TK, DIM), lambda qi, ki: (ki, 0)),
        ],
        out_specs=[
            pl.BlockSpec((TQ, 1), lambda qi, ki: (qi, 0)),
            pl.BlockSpec((TQ, 1), lambda qi, ki: (qi, 0)),
        ],
        out_shape=[
            jax.ShapeDtypeStruct((QP, 1), jnp.float32),
            jax.ShapeDtypeStruct((QP, 1), jnp.int32),
        ],
        scratch_shapes=[pltpu.VMEM((N_MEM // TK, 1, TK), jnp.float32)],
        interpret=interpret,
    )(q, qsq, mem_keys)

    dists = dists.reshape(OH, OWP)[:, :OW].reshape(Q)
    idx = idx.reshape(OH, OWP)[:, :OW].reshape(Q)
    return dists, idx


# speedup vs baseline: 1.3789x; 1.3789x over previous
import functools

import jax
import jax.numpy as jnp
from jax.experimental import pallas as pl
from jax.experimental.pallas import tpu as pltpu

KH, KW = 32, 32
STRIDE = 4
PAD = 10
H, W, C = 224, 224, 3
DIM = KH * KW * C
N_MEM = 4096
OH = (H + 2 * PAD - KH) // STRIDE + 1
OW = (W + 2 * PAD - KW) // STRIDE + 1
Q = OH * OW
U = (H + 2 * PAD) // STRIDE
AR = KH // STRIDE
G = C * KH
OWP = 56
QP = OH * OWP

TQ = 1008
TK = 512
KC = 512


def _unfold_kernel(img_ref, q_ref, qsq_ref):
    i = pl.program_id(0)
    blk = img_ref[:, pl.ds(i, AR), :, :]
    rm = blk.reshape(G, H + 2 * PAD)
    parts = [rm[:, 4 * min(j, OW - 1) : 4 * min(j, OW - 1) + KW]
             for j in range(OWP)]
    q3 = jnp.stack(parts, axis=0)
    q_t = q3.reshape(OWP, DIM)
    q_ref[0] = q_t
    qsq_ref[0] = jnp.sum(q_t * q_t, axis=1, keepdims=True)


def _search_kernel(q_ref, qsq_ref, k_ref, dist_ref, idx_ref, ksq_ref):
    qi = pl.program_id(0)
    ki = pl.program_id(1)
    kt = k_ref[...]

    @pl.when(qi == 0)
    def _fill_ksq():
        ksq_ref[ki] = jnp.sum(kt * kt, axis=1)[None, :]

    ksq = ksq_ref[ki]
    dot = jnp.zeros((TQ, TK), jnp.float32)
    for c in range(0, DIM, KC):
        dot = dot + jax.lax.dot_general(
            q_ref[:, c : c + KC], kt[:, c : c + KC],
            dimension_numbers=(((1,), (1,)), ((), ())),
            precision=jax.lax.Precision.DEFAULT,
            preferred_element_type=jnp.float32,
        )
    d = (qsq_ref[...] + ksq) - 2.0 * dot
    m = jnp.min(d, axis=1, keepdims=True)
    iota = jax.lax.broadcasted_iota(jnp.int32, d.shape, 1) + ki * TK
    am = jnp.min(jnp.where(d == m, iota, jnp.int32(2**30)), axis=1,
                 keepdims=True)

    @pl.when(ki == 0)
    def _init():
        dist_ref[...] = m
        idx_ref[...] = am

    @pl.when(ki != 0)
    def _update():
        better = m < dist_ref[...]
        idx_ref[...] = jnp.where(better, am, idx_ref[...])
        dist_ref[...] = jnp.where(better, m, dist_ref[...])


@functools.partial(jax.jit, static_argnames=("interpret",))
def kernel(image, mem_keys, interpret=False):
    img = jnp.transpose(image, (2, 0, 1))
    img = jnp.pad(img, ((0, 0), (PAD, PAD), (PAD, PAD)))
    img4 = img.reshape(C, U, STRIDE, H + 2 * PAD)

    q, qsq = pl.pallas_call(
        _unfold_kernel,
        grid=(OH,),
        in_specs=[pl.BlockSpec(img4.shape, lambda i: (0, 0, 0, 0))],
        out_specs=[
            pl.BlockSpec((1, OWP, DIM), lambda i: (i, 0, 0)),
            pl.BlockSpec((1, OWP, 1), lambda i: (i, 0, 0)),
        ],
        out_shape=[
            jax.ShapeDtypeStruct((OH, OWP, DIM), jnp.float32),
            jax.ShapeDtypeStruct((OH, OWP, 1), jnp.float32),
        ],
        interpret=interpret,
    )(img4)
    q = q.reshape(QP, DIM)
    qsq = qsq.reshape(QP, 1)

    dists, idx = pl.pallas_call(
        _search_kernel,
        grid=(QP // TQ, N_MEM // TK),
        in_specs=[
            pl.BlockSpec((TQ, DIM), lambda qi, ki: (qi, 0)),
            pl.BlockSpec((TQ, 1), lambda qi, ki: (qi, 0)),
            pl.BlockSpec((TK, DIM), lambda qi, ki: (ki, 0)),
        ],
        out_specs=[
            pl.BlockSpec((TQ, 1), lambda qi, ki: (qi, 0)),
            pl.BlockSpec((TQ, 1), lambda qi, ki: (qi, 0)),
        ],
        out_shape=[
            jax.ShapeDtypeStruct((QP, 1), jnp.float32),
            jax.ShapeDtypeStruct((QP, 1), jnp.int32),
        ],
        scratch_shapes=[pltpu.VMEM((N_MEM // TK, 1, TK), jnp.float32)],
        interpret=interpret,
    )(q, qsq, mem_keys)

    dists = dists.reshape(OH, OWP)[:, :OW].reshape(Q)
    idx = idx.reshape(OH, OWP)[:, :OW].reshape(Q)
    return dists, idx


# speedup vs baseline: 1.7436x; 1.2645x over previous
import functools

import jax
import jax.numpy as jnp
from jax.experimental import pallas as pl
from jax.experimental.pallas import tpu as pltpu

KH, KW = 32, 32
STRIDE = 4
PAD = 10
H, W, C = 224, 224, 3
DIM = KH * KW * C
N_MEM = 4096
OH = (H + 2 * PAD - KH) // STRIDE + 1
OW = (W + 2 * PAD - KW) // STRIDE + 1
Q = OH * OW
U = (H + 2 * PAD) // STRIDE
AR = KH // STRIDE
G = C * KH
OWP = 56
QP = OH * OWP

TQ = 1008
TK = 512
KC = 512


ROWS_PER_STEP = 3


def _unfold_kernel(img_ref, q_ref, qsq_ref):
    i = pl.program_id(0)
    blk = img_ref[:, pl.ds(ROWS_PER_STEP * i, AR + ROWS_PER_STEP - 1), :, :]
    for a in range(ROWS_PER_STEP):
        rm = blk[:, a : a + AR].reshape(G, H + 2 * PAD)
        parts = [rm[:, 4 * min(j, OW - 1) : 4 * min(j, OW - 1) + KW]
                 for j in range(OWP)]
        q3 = jnp.stack(parts, axis=0)
        q_t = q3.reshape(OWP, DIM)
        q_ref[a] = q_t
        qsq_ref[a] = jnp.sum(q_t * q_t, axis=1, keepdims=True)


def _search_kernel(q_ref, qsq_ref, k_ref, dist_ref, idx_ref, ksq_ref):
    qi = pl.program_id(0)
    ki = pl.program_id(1)
    kt = k_ref[...]

    @pl.when(qi == 0)
    def _fill_ksq():
        ksq_ref[ki] = jnp.sum(kt * kt, axis=1)[None, :]

    ksq = ksq_ref[ki]
    dot = jnp.zeros((TQ, TK), jnp.float32)
    for c in range(0, DIM, KC):
        dot = dot + jax.lax.dot_general(
            q_ref[:, c : c + KC], kt[:, c : c + KC],
            dimension_numbers=(((1,), (1,)), ((), ())),
            precision=jax.lax.Precision.DEFAULT,
            preferred_element_type=jnp.float32,
        )
    d = (qsq_ref[...] + ksq) - 2.0 * dot
    m = jnp.min(d, axis=1, keepdims=True)
    iota = jax.lax.broadcasted_iota(jnp.int32, d.shape, 1) + ki * TK
    am = jnp.min(jnp.where(d == m, iota, jnp.int32(2**30)), axis=1,
                 keepdims=True)

    @pl.when(ki == 0)
    def _init():
        dist_ref[...] = m
        idx_ref[...] = am

    @pl.when(ki != 0)
    def _update():
        better = m < dist_ref[...]
        idx_ref[...] = jnp.where(better, am, idx_ref[...])
        dist_ref[...] = jnp.where(better, m, dist_ref[...])


@functools.partial(jax.jit, static_argnames=("interpret",))
def kernel(image, mem_keys, interpret=False):
    img = jnp.transpose(image, (2, 0, 1))
    img = jnp.pad(img, ((0, 0), (PAD, PAD), (PAD, PAD)))
    img4 = img.reshape(C, U, STRIDE, H + 2 * PAD)

    q, qsq = pl.pallas_call(
        _unfold_kernel,
        grid=(OH // ROWS_PER_STEP,),
        in_specs=[pl.BlockSpec(img4.shape, lambda i: (0, 0, 0, 0))],
        out_specs=[
            pl.BlockSpec((ROWS_PER_STEP, OWP, DIM), lambda i: (i, 0, 0)),
            pl.BlockSpec((ROWS_PER_STEP, OWP, 1), lambda i: (i, 0, 0)),
        ],
        out_shape=[
            jax.ShapeDtypeStruct((OH, OWP, DIM), jnp.float32),
            jax.ShapeDtypeStruct((OH, OWP, 1), jnp.float32),
        ],
        interpret=interpret,
    )(img4)
    q = q.reshape(QP, DIM)
    qsq = qsq.reshape(QP, 1)

    dists, idx = pl.pallas_call(
        _search_kernel,
        grid=(QP // TQ, N_MEM // TK),
        in_specs=[
            pl.BlockSpec((TQ, DIM), lambda qi, ki: (qi, 0)),
            pl.BlockSpec((TQ, 1), lambda qi, ki: (qi, 0)),
            pl.BlockSpec((TK, DIM), lambda qi, ki: (ki, 0)),
        ],
        out_specs=[
            pl.BlockSpec((TQ, 1), lambda qi, ki: (qi, 0)),
            pl.BlockSpec((TQ, 1), lambda qi, ki: (qi, 0)),
        ],
        out_shape=[
            jax.ShapeDtypeStruct((QP, 1), jnp.float32),
            jax.ShapeDtypeStruct((QP, 1), jnp.int32),
        ],
        scratch_shapes=[pltpu.VMEM((N_MEM // TK, 1, TK), jnp.float32)],
        interpret=interpret,
    )(q, qsq, mem_keys)

    dists = dists.reshape(OH, OWP)[:, :OW].reshape(Q)
    idx = idx.reshape(OH, OWP)[:, :OW].reshape(Q)
    return dists, idx


# speedup vs baseline: 1.8516x; 1.0620x over previous
import functools

import jax
import jax.numpy as jnp
from jax.experimental import pallas as pl
from jax.experimental.pallas import tpu as pltpu

KH, KW = 32, 32
STRIDE = 4
PAD = 10
H, W, C = 224, 224, 3
DIM = KH * KW * C
N_MEM = 4096
OH = (H + 2 * PAD - KH) // STRIDE + 1
OW = (W + 2 * PAD - KW) // STRIDE + 1
Q = OH * OW
U = (H + 2 * PAD) // STRIDE
AR = KH // STRIDE
G = C * KH
OWP = 56
QP = OH * OWP

TQ = 1008
TK = 512
KC = 512


ROWS_PER_STEP = 6


def _unfold_kernel(img_ref, q_ref, qsq_ref):
    i = pl.program_id(0)
    blk = img_ref[:, pl.ds(ROWS_PER_STEP * i, AR + ROWS_PER_STEP - 1), :, :]
    for a in range(ROWS_PER_STEP):
        rm = blk[:, a : a + AR].reshape(G, H + 2 * PAD)
        parts = [rm[:, 4 * min(j, OW - 1) : 4 * min(j, OW - 1) + KW]
                 for j in range(OWP)]
        q3 = jnp.stack(parts, axis=0)
        q_t = q3.reshape(OWP, DIM)
        q_ref[a] = q_t
        qsq_ref[a] = jnp.sum(q_t * q_t, axis=1, keepdims=True)


def _search_kernel(q_ref, qsq_ref, k_ref, dist_ref, idx_ref, ksq_ref):
    qi = pl.program_id(0)
    ki = pl.program_id(1)
    kt = k_ref[...]

    @pl.when(qi == 0)
    def _fill_ksq():
        ksq_ref[ki] = jnp.sum(kt * kt, axis=1)[None, :]

    ksq = ksq_ref[ki]
    dot = jnp.zeros((TQ, TK), jnp.float32)
    for c in range(0, DIM, KC):
        dot = dot + jax.lax.dot_general(
            q_ref[:, c : c + KC], kt[:, c : c + KC],
            dimension_numbers=(((1,), (1,)), ((), ())),
            precision=jax.lax.Precision.DEFAULT,
            preferred_element_type=jnp.float32,
        )
    d = (qsq_ref[...] + ksq) - 2.0 * dot
    m = jnp.min(d, axis=1, keepdims=True)
    iota = jax.lax.broadcasted_iota(jnp.int32, d.shape, 1) + ki * TK
    am = jnp.min(jnp.where(d == m, iota, jnp.int32(2**30)), axis=1,
                 keepdims=True)

    @pl.when(ki == 0)
    def _init():
        dist_ref[...] = m
        idx_ref[...] = am

    @pl.when(ki != 0)
    def _update():
        better = m < dist_ref[...]
        idx_ref[...] = jnp.where(better, am, idx_ref[...])
        dist_ref[...] = jnp.where(better, m, dist_ref[...])


@functools.partial(jax.jit, static_argnames=("interpret",))
def kernel(image, mem_keys, interpret=False):
    img = jnp.transpose(image, (2, 0, 1))
    img = jnp.pad(img, ((0, 0), (PAD, PAD), (PAD, PAD)))
    img4 = img.reshape(C, U, STRIDE, H + 2 * PAD)

    q, qsq = pl.pallas_call(
        _unfold_kernel,
        grid=(OH // ROWS_PER_STEP,),
        in_specs=[pl.BlockSpec(img4.shape, lambda i: (0, 0, 0, 0))],
        out_specs=[
            pl.BlockSpec((ROWS_PER_STEP, OWP, DIM), lambda i: (i, 0, 0)),
            pl.BlockSpec((ROWS_PER_STEP, OWP, 1), lambda i: (i, 0, 0)),
        ],
        out_shape=[
            jax.ShapeDtypeStruct((OH, OWP, DIM), jnp.float32),
            jax.ShapeDtypeStruct((OH, OWP, 1), jnp.float32),
        ],
        interpret=interpret,
    )(img4)
    q = q.reshape(QP, DIM)
    qsq = qsq.reshape(QP, 1)

    dists, idx = pl.pallas_call(
        _search_kernel,
        grid=(QP // TQ, N_MEM // TK),
        in_specs=[
            pl.BlockSpec((TQ, DIM), lambda qi, ki: (qi, 0)),
            pl.BlockSpec((TQ, 1), lambda qi, ki: (qi, 0)),
            pl.BlockSpec((TK, DIM), lambda qi, ki: (ki, 0)),
        ],
        out_specs=[
            pl.BlockSpec((TQ, 1), lambda qi, ki: (qi, 0)),
            pl.BlockSpec((TQ, 1), lambda qi, ki: (qi, 0)),
        ],
        out_shape=[
            jax.ShapeDtypeStruct((QP, 1), jnp.float32),
            jax.ShapeDtypeStruct((QP, 1), jnp.int32),
        ],
        scratch_shapes=[pltpu.VMEM((N_MEM // TK, 1, TK), jnp.float32)],
        interpret=interpret,
    )(q, qsq, mem_keys)

    dists = dists.reshape(OH, OWP)[:, :OW].reshape(Q)
    idx = idx.reshape(OH, OWP)[:, :OW].reshape(Q)
    return dists, idx


# speedup vs baseline: 1.8809x; 1.0158x over previous
import functools

import jax
import jax.numpy as jnp
from jax.experimental import pallas as pl
from jax.experimental.pallas import tpu as pltpu

KH, KW = 32, 32
STRIDE = 4
PAD = 10
H, W, C = 224, 224, 3
DIM = KH * KW * C
N_MEM = 4096
OH = (H + 2 * PAD - KH) // STRIDE + 1
OW = (W + 2 * PAD - KW) // STRIDE + 1
Q = OH * OW
U = (H + 2 * PAD) // STRIDE
AR = KH // STRIDE
G = C * KH
OWP = 56
QP = OH * OWP

TQ = 1008
TK = 512
KC = 512


ROWS_PER_STEP = 9


def _unfold_kernel(img_ref, q_ref, qsq_ref):
    i = pl.program_id(0)
    blk = img_ref[:, pl.ds(ROWS_PER_STEP * i, AR + ROWS_PER_STEP - 1), :, :]
    for a in range(ROWS_PER_STEP):
        rm = blk[:, a : a + AR].reshape(G, H + 2 * PAD)
        parts = [rm[:, 4 * min(j, OW - 1) : 4 * min(j, OW - 1) + KW]
                 for j in range(OWP)]
        q3 = jnp.stack(parts, axis=0)
        q_t = q3.reshape(OWP, DIM)
        q_ref[a] = q_t
        qsq_ref[a] = jnp.sum(q_t * q_t, axis=1, keepdims=True)


def _search_kernel(q_ref, qsq_ref, k_ref, dist_ref, idx_ref, ksq_ref):
    qi = pl.program_id(0)
    ki = pl.program_id(1)
    kt = k_ref[...]

    @pl.when(qi == 0)
    def _fill_ksq():
        ksq_ref[ki] = jnp.sum(kt * kt, axis=1)[None, :]

    ksq = ksq_ref[ki]
    dot = jnp.zeros((TQ, TK), jnp.float32)
    for c in range(0, DIM, KC):
        dot = dot + jax.lax.dot_general(
            q_ref[:, c : c + KC], kt[:, c : c + KC],
            dimension_numbers=(((1,), (1,)), ((), ())),
            precision=jax.lax.Precision.DEFAULT,
            preferred_element_type=jnp.float32,
        )
    d = (qsq_ref[...] + ksq) - 2.0 * dot
    m = jnp.min(d, axis=1, keepdims=True)
    iota = jax.lax.broadcasted_iota(jnp.int32, d.shape, 1) + ki * TK
    am = jnp.min(jnp.where(d == m, iota, jnp.int32(2**30)), axis=1,
                 keepdims=True)

    @pl.when(ki == 0)
    def _init():
        dist_ref[...] = m
        idx_ref[...] = am

    @pl.when(ki != 0)
    def _update():
        better = m < dist_ref[...]
        idx_ref[...] = jnp.where(better, am, idx_ref[...])
        dist_ref[...] = jnp.where(better, m, dist_ref[...])


@functools.partial(jax.jit, static_argnames=("interpret",))
def kernel(image, mem_keys, interpret=False):
    img = jnp.transpose(image, (2, 0, 1))
    img = jnp.pad(img, ((0, 0), (PAD, PAD), (PAD, PAD)))
    img4 = img.reshape(C, U, STRIDE, H + 2 * PAD)

    q, qsq = pl.pallas_call(
        _unfold_kernel,
        grid=(OH // ROWS_PER_STEP,),
        in_specs=[pl.BlockSpec(img4.shape, lambda i: (0, 0, 0, 0))],
        out_specs=[
            pl.BlockSpec((ROWS_PER_STEP, OWP, DIM), lambda i: (i, 0, 0)),
            pl.BlockSpec((ROWS_PER_STEP, OWP, 1), lambda i: (i, 0, 0)),
        ],
        out_shape=[
            jax.ShapeDtypeStruct((OH, OWP, DIM), jnp.float32),
            jax.ShapeDtypeStruct((OH, OWP, 1), jnp.float32),
        ],
        interpret=interpret,
    )(img4)
    q = q.reshape(QP, DIM)
    qsq = qsq.reshape(QP, 1)

    dists, idx = pl.pallas_call(
        _search_kernel,
        grid=(QP // TQ, N_MEM // TK),
        in_specs=[
            pl.BlockSpec((TQ, DIM), lambda qi, ki: (qi, 0)),
            pl.BlockSpec((TQ, 1), lambda qi, ki: (qi, 0)),
            pl.BlockSpec((TK, DIM), lambda qi, ki: (ki, 0)),
        ],
        out_specs=[
            pl.BlockSpec((TQ, 1), lambda qi, ki: (qi, 0)),
            pl.BlockSpec((TQ, 1), lambda qi, ki: (qi, 0)),
        ],
        out_shape=[
            jax.ShapeDtypeStruct((QP, 1), jnp.float32),
            jax.ShapeDtypeStruct((QP, 1), jnp.int32),
        ],
        scratch_shapes=[pltpu.VMEM((N_MEM // TK, 1, TK), jnp.float32)],
        interpret=interpret,
    )(q, qsq, mem_keys)

    dists = dists.reshape(OH, OWP)[:, :OW].reshape(Q)
    idx = idx.reshape(OH, OWP)[:, :OW].reshape(Q)
    return dists, idx


# speedup vs baseline: 1.8899x; 1.0048x over previous
import functools

import jax
import jax.numpy as jnp
from jax.experimental import pallas as pl
from jax.experimental.pallas import tpu as pltpu

KH, KW = 32, 32
STRIDE = 4
PAD = 10
H, W, C = 224, 224, 3
DIM = KH * KW * C
N_MEM = 4096
OH = (H + 2 * PAD - KH) // STRIDE + 1
OW = (W + 2 * PAD - KW) // STRIDE + 1
Q = OH * OW
U = (H + 2 * PAD) // STRIDE
AR = KH // STRIDE
G = C * KH
OWP = 56
QP = OH * OWP

TQ = 1008
TK = 512
KC = 512


ROWS_PER_STEP = 9


def _unfold_kernel(img_ref, q_ref, qsq_ref):
    i = pl.program_id(0)
    blk = img_ref[:, pl.ds(ROWS_PER_STEP * i, AR + ROWS_PER_STEP - 1), :, :]
    for a in range(ROWS_PER_STEP):
        rm = blk[:, a : a + AR].reshape(G, H + 2 * PAD)
        parts = [rm[:, 4 * min(j, OW - 1) : 4 * min(j, OW - 1) + KW]
                 for j in range(OWP)]
        q3 = jnp.stack(parts, axis=0)
        q_t = q3.reshape(OWP, DIM)
        q_ref[a] = q_t
        qsq_ref[a] = jnp.sum(q_t * q_t, axis=1, keepdims=True)


def _search_kernel(q_ref, qsq_ref, k_ref, dist_ref, idx_ref, ksq_ref):
    qi = pl.program_id(0)
    ki = pl.program_id(1)
    kt = k_ref[...]

    @pl.when(qi == 0)
    def _fill_ksq():
        ksq_ref[pl.ds(ki, 1), :] = jnp.sum(kt * kt, axis=1)[None, :]

    ksq = ksq_ref[pl.ds(ki, 1), :]
    dot = jnp.zeros((TQ, TK), jnp.float32)
    for c in range(0, DIM, KC):
        dot = dot + jax.lax.dot_general(
            q_ref[:, c : c + KC], kt[:, c : c + KC],
            dimension_numbers=(((1,), (1,)), ((), ())),
            precision=jax.lax.Precision.DEFAULT,
            preferred_element_type=jnp.float32,
        )
    d = (qsq_ref[...] + ksq) - 2.0 * dot
    m = jnp.min(d, axis=1, keepdims=True)
    iota = jax.lax.broadcasted_iota(jnp.int32, d.shape, 1)
    am = jnp.min(jnp.where(d == m, iota, jnp.int32(2**30)), axis=1,
                 keepdims=True) + ki * TK

    @pl.when(ki == 0)
    def _init():
        dist_ref[...] = m
        idx_ref[...] = am

    @pl.when(ki != 0)
    def _update():
        better = m < dist_ref[...]
        idx_ref[...] = jnp.where(better, am, idx_ref[...])
        dist_ref[...] = jnp.where(better, m, dist_ref[...])


@functools.partial(jax.jit, static_argnames=("interpret",))
def kernel(image, mem_keys, interpret=False):
    img = jnp.transpose(image, (2, 0, 1))
    img = jnp.pad(img, ((0, 0), (PAD, PAD), (PAD, PAD)))
    img4 = img.reshape(C, U, STRIDE, H + 2 * PAD)

    q, qsq = pl.pallas_call(
        _unfold_kernel,
        grid=(OH // ROWS_PER_STEP,),
        in_specs=[pl.BlockSpec(img4.shape, lambda i: (0, 0, 0, 0))],
        out_specs=[
            pl.BlockSpec((ROWS_PER_STEP, OWP, DIM), lambda i: (i, 0, 0)),
            pl.BlockSpec((ROWS_PER_STEP, OWP, 1), lambda i: (i, 0, 0)),
        ],
        out_shape=[
            jax.ShapeDtypeStruct((OH, OWP, DIM), jnp.float32),
            jax.ShapeDtypeStruct((OH, OWP, 1), jnp.float32),
        ],
        interpret=interpret,
    )(img4)
    q = q.reshape(QP, DIM)
    qsq = qsq.reshape(QP, 1)

    dists, idx = pl.pallas_call(
        _search_kernel,
        grid=(QP // TQ, N_MEM // TK),
        in_specs=[
            pl.BlockSpec((TQ, DIM), lambda qi, ki: (qi, 0)),
            pl.BlockSpec((TQ, 1), lambda qi, ki: (qi, 0)),
            pl.BlockSpec((TK, DIM), lambda qi, ki: (ki, 0)),
        ],
        out_specs=[
            pl.BlockSpec((TQ, 1), lambda qi, ki: (qi, 0)),
            pl.BlockSpec((TQ, 1), lambda qi, ki: (qi, 0)),
        ],
        out_shape=[
            jax.ShapeDtypeStruct((QP, 1), jnp.float32),
            jax.ShapeDtypeStruct((QP, 1), jnp.int32),
        ],
        scratch_shapes=[pltpu.VMEM((N_MEM // TK, TK), jnp.float32)],
        interpret=interpret,
    )(q, qsq, mem_keys)

    dists = dists.reshape(OH, OWP)[:, :OW].reshape(Q)
    idx = idx.reshape(OH, OWP)[:, :OW].reshape(Q)
    return dists, idx
